# baseline (speedup 1.0000x reference)
import functools

import jax
import jax.numpy as jnp
from jax import lax
from jax.experimental import pallas as pl
from jax.experimental.pallas import tpu as pltpu

NB2 = 8


def _vq_body(w_ref, z_ref, cb_ref, zq_ref, loss_ref, cbw_s, nb_s, rev_s, *,
             size, inv_bs):
    step = pl.program_id(0)
    w = w_ref[0, 0]
    cb = cb_ref[...]

    @pl.when(step == 0)
    def _prep():
        cbw_s[...] = cb * (2.0 * w)
        nb_s[...] = (-w) * jnp.sum(cb * cb, axis=1)[None, :]
        rev_s[...] = (jnp.int32(size) - lax.broadcasted_iota(
            jnp.int32, (1, size), 1)).astype(jnp.float32)

    rev = rev_s[...]
    loss = jnp.float32(0.0)
    for i in range(NB2):
        zb = z_ref[i]
        g = lax.dot_general(zb, cbw_s[...], (((0,), (1,)), ((), ())),
                            preferred_element_type=jnp.float32) + nb_s[...]

        rowmax = jnp.max(g, axis=1)
        e = jnp.exp(g - rowmax[:, None])
        denom = jnp.sum(e, axis=1)
        sumeg = jnp.sum(e * g, axis=1)
        kld = jnp.sum(sumeg / denom - rowmax - jnp.log(denom))

        vmax = jnp.max(jnp.where(g == rowmax[:, None], rev, 0.0), axis=1)
        onehot = (rev == vmax[:, None]).astype(jnp.float32)
        zq_ref[i] = lax.dot_general(cb, onehot, (((0,), (1,)), ((), ())),
                                    preferred_element_type=jnp.float32)

        loss += kld + w * jnp.sum(zb * zb) - jnp.sum(rowmax)

    @pl.when(step == 0)
    def _init():
        loss_ref[0, 0] = 0.0

    loss_ref[0, 0] += loss * inv_bs


def kernel(z, codebook, var_q, var_init):
    bs, dim_z, d1, d2 = z.shape
    size, _ = codebook.shape
    npix = d1 * d2
    z3 = z.reshape(bs, dim_z, npix)

    var_q_eff = jax.nn.sigmoid(var_q) * 2.0 * var_init
    w = (0.5 / jnp.clip(var_q_eff, 1e-10, None)).reshape(1, 1)

    body = functools.partial(_vq_body, size=size, inv_bs=1.0 / bs)
    zq3, loss = pl.pallas_call(
        body,
        grid=(bs // NB2,),
        in_specs=[
            pl.BlockSpec(memory_space=pltpu.SMEM),
            pl.BlockSpec((NB2, dim_z, npix), lambda s: (s, 0, 0)),
            pl.BlockSpec((size, dim_z), lambda s: (0, 0)),
        ],
        out_specs=[
            pl.BlockSpec((NB2, dim_z, npix), lambda s: (s, 0, 0)),
            pl.BlockSpec(memory_space=pltpu.SMEM),
        ],
        out_shape=[
            jax.ShapeDtypeStruct((bs, dim_z, npix), jnp.float32),
            jax.ShapeDtypeStruct((1, 1), jnp.float32),
        ],
        scratch_shapes=[
            pltpu.VMEM((size, dim_z), jnp.float32),
            pltpu.VMEM((1, size), jnp.float32),
            pltpu.VMEM((1, size), jnp.float32),
        ],
    )(w, z3, codebook)
    return zq3.reshape(bs, dim_z, d1, d2), loss[0, 0]


# speedup vs baseline: 1.0472x; 1.0472x over previous
import functools

import jax
import jax.numpy as jnp
from jax import lax
from jax.experimental import pallas as pl
from jax.experimental.pallas import tpu as pltpu

NB2 = 4


def _vq_body(w_ref, z_ref, cb_ref, zq_ref, loss_ref, cbe_s, ze_s, rev_s, *,
             size, dim, npix, inv_bs):
    step = pl.program_id(0)
    w = w_ref[0, 0]
    cb = cb_ref[...]

    @pl.when(step == 0)
    def _prep():
        cbe_s[:, :dim] = cb * (2.0 * w)
        cbe_s[:, dim:] = (-w) * jnp.sum(cb * cb, axis=1)[:, None]
        ze_s[dim:, :] = jnp.ones((1, npix), jnp.float32)
        rev_s[...] = (jnp.int32(size) - lax.broadcasted_iota(
            jnp.int32, (1, size), 1)).astype(jnp.float32)

    rev = rev_s[...]
    loss = jnp.float32(0.0)
    for i in range(NB2):
        zb = z_ref[i]
        ze_s[:dim, :] = zb
        g = lax.dot_general(ze_s[...], cbe_s[...], (((0,), (1,)), ((), ())),
                            preferred_element_type=jnp.float32)

        rowmax = jnp.max(g, axis=1)
        e = jnp.exp(g - rowmax[:, None])
        denom = jnp.sum(e, axis=1)
        sumeg = jnp.sum(e * g, axis=1)
        kld = jnp.sum(sumeg / denom - rowmax - jnp.log(denom))

        vmax = jnp.max(jnp.where(g == rowmax[:, None], rev, 0.0), axis=1)
        onehot = (rev == vmax[:, None]).astype(jnp.float32)
        zq_ref[i] = lax.dot_general(cb, onehot, (((0,), (1,)), ((), ())),
                                    preferred_element_type=jnp.float32)

        loss += kld + w * jnp.sum(zb * zb) - jnp.sum(rowmax)

    @pl.when(step == 0)
    def _init():
        loss_ref[0, 0] = 0.0

    loss_ref[0, 0] += loss * inv_bs


def kernel(z, codebook, var_q, var_init):
    bs, dim_z, d1, d2 = z.shape
    size, _ = codebook.shape
    npix = d1 * d2
    z3 = z.reshape(bs, dim_z, npix)

    var_q_eff = jax.nn.sigmoid(var_q) * 2.0 * var_init
    w = (0.5 / jnp.clip(var_q_eff, 1e-10, None)).reshape(1, 1)

    body = functools.partial(_vq_body, size=size, dim=dim_z, npix=npix,
                             inv_bs=1.0 / bs)
    zq3, loss = pl.pallas_call(
        body,
        grid=(bs // NB2,),
        in_specs=[
            pl.BlockSpec(memory_space=pltpu.SMEM),
            pl.BlockSpec((NB2, dim_z, npix), lambda s: (s, 0, 0)),
            pl.BlockSpec((size, dim_z), lambda s: (0, 0)),
        ],
        out_specs=[
            pl.BlockSpec((NB2, dim_z, npix), lambda s: (s, 0, 0)),
            pl.BlockSpec(memory_space=pltpu.SMEM),
        ],
        out_shape=[
            jax.ShapeDtypeStruct((bs, dim_z, npix), jnp.float32),
            jax.ShapeDtypeStruct((1, 1), jnp.float32),
        ],
        scratch_shapes=[
            pltpu.VMEM((size, dim_z + 1), jnp.float32),
            pltpu.VMEM((dim_z + 1, npix), jnp.float32),
            pltpu.VMEM((1, size), jnp.float32),
        ],
    )(w, z3, codebook)
    return zq3.reshape(bs, dim_z, d1, d2), loss[0, 0]


# speedup vs baseline: 1.0668x; 1.0187x over previous
import functools

import jax
import jax.numpy as jnp
from jax import lax
from jax.experimental import pallas as pl
from jax.experimental.pallas import tpu as pltpu

NB2 = 4
KP = 128


def _vq_body(w_ref, z_ref, cb_ref, zq_ref, loss_ref, cbe_s, ze_s, rev_s, *,
             size, dim, npix, inv_bs):
    step = pl.program_id(0)
    w = w_ref[0, 0]
    cb = cb_ref[...]

    @pl.when(step == 0)
    def _prep():
        cbe_s[:, :dim] = cb * (2.0 * w)
        cbe_s[:, dim:dim + 1] = (-w) * jnp.sum(cb * cb, axis=1)[:, None]
        cbe_s[:, dim + 1:] = jnp.zeros((size, KP - dim - 1), jnp.float32)
        ze_s[dim:dim + 1, :] = jnp.ones((1, npix), jnp.float32)
        ze_s[dim + 1:, :] = jnp.zeros((KP - dim - 1, npix), jnp.float32)
        rev_s[...] = (jnp.int32(size) - lax.broadcasted_iota(
            jnp.int32, (1, size), 1)).astype(jnp.float32)

    rev = rev_s[...]
    loss = jnp.float32(0.0)
    for i in range(NB2):
        zb = z_ref[i]
        ze_s[:dim, :] = zb
        g = lax.dot_general(ze_s[...], cbe_s[...], (((0,), (1,)), ((), ())),
                            preferred_element_type=jnp.float32)

        rowmax = jnp.max(g, axis=1)
        e = jnp.exp(g - rowmax[:, None])
        denom = jnp.sum(e, axis=1)
        sumeg = jnp.sum(e * g, axis=1)
        kld = jnp.sum(sumeg / denom - rowmax - jnp.log(denom))

        vmax = jnp.max(jnp.where(g == rowmax[:, None], rev, 0.0), axis=1)
        onehot = (rev == vmax[:, None]).astype(jnp.float32)
        zq_ref[i] = lax.dot_general(cb, onehot, (((0,), (1,)), ((), ())),
                                    preferred_element_type=jnp.float32)

        loss += kld + w * jnp.sum(zb * zb) - jnp.sum(rowmax)

    @pl.when(step == 0)
    def _init():
        loss_ref[0, 0] = 0.0

    loss_ref[0, 0] += loss * inv_bs


def kernel(z, codebook, var_q, var_init):
    bs, dim_z, d1, d2 = z.shape
    size, _ = codebook.shape
    npix = d1 * d2
    z3 = z.reshape(bs, dim_z, npix)

    var_q_eff = jax.nn.sigmoid(var_q) * 2.0 * var_init
    w = (0.5 / jnp.clip(var_q_eff, 1e-10, None)).reshape(1, 1)

    body = functools.partial(_vq_body, size=size, dim=dim_z, npix=npix,
                             inv_bs=1.0 / bs)
    zq3, loss = pl.pallas_call(
        body,
        grid=(bs // NB2,),
        in_specs=[
            pl.BlockSpec(memory_space=pltpu.SMEM),
            pl.BlockSpec((NB2, dim_z, npix), lambda s: (s, 0, 0)),
            pl.BlockSpec((size, dim_z), lambda s: (0, 0)),
        ],
        out_specs=[
            pl.BlockSpec((NB2, dim_z, npix), lambda s: (s, 0, 0)),
            pl.BlockSpec(memory_space=pltpu.SMEM),
        ],
        out_shape=[
            jax.ShapeDtypeStruct((bs, dim_z, npix), jnp.float32),
            jax.ShapeDtypeStruct((1, 1), jnp.float32),
        ],
        scratch_shapes=[
            pltpu.VMEM((size, KP), jnp.float32),
            pltpu.VMEM((KP, npix), jnp.float32),
            pltpu.VMEM((1, size), jnp.float32),
        ],
    )(w, z3, codebook)
    return zq3.reshape(bs, dim_z, d1, d2), loss[0, 0]


# speedup vs baseline: 1.0694x; 1.0024x over previous
import functools

import jax
import jax.numpy as jnp
from jax import lax
from jax.experimental import pallas as pl
from jax.experimental.pallas import tpu as pltpu

NB2 = 4
KP = 128


def _vq_body(w_ref, z_ref, cb_ref, zq_ref, loss_ref, cbe_s, rev_s, *,
             size, dim, npix, inv_bs):
    step = pl.program_id(0)
    w = w_ref[0, 0]
    cb = cb_ref[...]

    @pl.when(step == 0)
    def _prep():
        cbe_s[:, :dim] = cb * (2.0 * w)
        cbe_s[:, dim:dim + 1] = (-w) * jnp.sum(cb * cb, axis=1)[:, None]
        cbe_s[:, dim + 1:] = jnp.zeros((size, KP - dim - 1), jnp.float32)
        rev_s[...] = (jnp.int32(size) - lax.broadcasted_iota(
            jnp.int32, (1, size), 1)).astype(jnp.float32)

    rev = rev_s[...]
    loss = jnp.float32(0.0)
    for i in range(NB2):
        zb = z_ref[i]
        ze = jnp.concatenate(
            [zb, jnp.ones((1, npix), jnp.float32),
             jnp.zeros((KP - dim - 1, npix), jnp.float32)], axis=0)
        g = lax.dot_general(ze, cbe_s[...], (((0,), (1,)), ((), ())),
                            preferred_element_type=jnp.float32)

        rowmax = jnp.max(g, axis=1)
        e = jnp.exp(g - rowmax[:, None])
        denom = jnp.sum(e, axis=1)
        sumeg = jnp.sum(e * g, axis=1)
        kld = jnp.sum(sumeg / denom - rowmax - jnp.log(denom))

        vmax = jnp.max(jnp.where(g == rowmax[:, None], rev, 0.0), axis=1)
        onehot = (rev == vmax[:, None]).astype(jnp.float32)
        zq_ref[i] = lax.dot_general(cb, onehot, (((0,), (1,)), ((), ())),
                                    preferred_element_type=jnp.float32)

        loss += kld + w * jnp.sum(zb * zb) - jnp.sum(rowmax)

    @pl.when(step == 0)
    def _init():
        loss_ref[0, 0] = 0.0

    loss_ref[0, 0] += loss * inv_bs


def kernel(z, codebook, var_q, var_init):
    bs, dim_z, d1, d2 = z.shape
    size, _ = codebook.shape
    npix = d1 * d2
    z3 = z.reshape(bs, dim_z, npix)

    var_q_eff = jax.nn.sigmoid(var_q) * 2.0 * var_init
    w = (0.5 / jnp.clip(var_q_eff, 1e-10, None)).reshape(1, 1)

    body = functools.partial(_vq_body, size=size, dim=dim_z, npix=npix,
                             inv_bs=1.0 / bs)
    zq3, loss = pl.pallas_call(
        body,
        grid=(bs // NB2,),
        in_specs=[
            pl.BlockSpec(memory_space=pltpu.SMEM),
            pl.BlockSpec((NB2, dim_z, npix), lambda s: (s, 0, 0)),
            pl.BlockSpec((size, dim_z), lambda s: (0, 0)),
        ],
        out_specs=[
            pl.BlockSpec((NB2, dim_z, npix), lambda s: (s, 0, 0)),
            pl.BlockSpec(memory_space=pltpu.SMEM),
        ],
        out_shape=[
            jax.ShapeDtypeStruct((bs, dim_z, npix), jnp.float32),
            jax.ShapeDtypeStruct((1, 1), jnp.float32),
        ],
        scratch_shapes=[
            pltpu.VMEM((size, KP), jnp.float32),
            pltpu.VMEM((1, size), jnp.float32),
        ],
    )(w, z3, codebook)
    return zq3.reshape(bs, dim_z, d1, d2), loss[0, 0]
